# baseline (speedup 1.0000x reference)
import functools

import jax
import jax.numpy as jnp
from jax import lax
from jax.experimental import pallas as pl
from jax.experimental.pallas import tpu as pltpu
from jax.experimental.pallas import tpu_sc as plsc

N = 10000
NP = 10240
F_IN = 128
F_HID = 128
F_OUT = 64
E = 320000
NC = 2
NS = 16
CH = 128
EPT = (E // (NS * CH)) * CH
KS = EPT // CH
KSL = (E - (NS - 1) * EPT) // CH
EPW = (E // (NC * NS * CH)) * CH
KW = EPW // CH
KWL = (E - (NC * NS - 1) * EPW) // CH
RPT = NP // NS
BR = NP // 8

_mesh = plsc.VectorSubcoreMesh(core_axis_name="c", subcore_axis_name="s")


@functools.partial(
    pl.kernel, mesh=_mesh,
    out_type=jax.ShapeDtypeStruct((NC, NP), jnp.float32),
    scratch_types=[
        pltpu.VMEM((2, CH), jnp.int32),
        pltpu.VMEM((CH,), jnp.float32),
        pltpu.VMEM_SHARED((NP,), jnp.float32),
        pltpu.SemaphoreType.DMA,
    ],
)
def _sc_degree(ei_hbm, zeros_hbm, out_hbm, idx2, ones_v, acc_sh, sem_i):
    cid = lax.axis_index("c")
    sid = lax.axis_index("s")
    w = cid * NS + sid
    base = w * EPW
    kw = jnp.where(w == NC * NS - 1, KWL, KW)
    for i in range(CH // 16):
        ones_v[pl.ds(i * 16, 16)] = jnp.ones((16,), jnp.float32)
    pltpu.sync_copy(zeros_hbm.at[pl.ds(sid * RPT, RPT)],
                    acc_sh.at[pl.ds(sid * RPT, RPT)])
    pltpu.sync_copy(ei_hbm.at[1, pl.ds(base, CH)], idx2.at[0])
    plsc.subcore_barrier()

    def body(j, carry):
        par = lax.rem(j, 2)

        @pl.when(j > 0)
        def _w():
            pltpu.make_async_copy(ei_hbm.at[1, pl.ds(base + j * CH, CH)],
                                  idx2.at[par], sem_i).wait()

        @pl.when(j + 1 < kw)
        def _pf():
            pltpu.async_copy(ei_hbm.at[1, pl.ds(base + (j + 1) * CH, CH)],
                             idx2.at[1 - par], sem_i)

        pltpu.sync_copy(ones_v, acc_sh.at[idx2.at[par]], add=True)
        return carry

    lax.fori_loop(0, kw, body, 0)
    plsc.subcore_barrier()
    pltpu.sync_copy(acc_sh.at[pl.ds(sid * RPT, RPT)],
                    out_hbm.at[cid, pl.ds(sid * RPT, RPT)])


def _make_sc_scatter(f2):
    @functools.partial(
        pl.kernel, mesh=_mesh,
        compiler_params=pltpu.CompilerParams(use_tc_tiling_on_sc=False),
        out_type=jax.ShapeDtypeStruct((NC, NP, f2), jnp.float32),
        scratch_types=[
            pltpu.VMEM((4, 2, CH), jnp.int32),
            pltpu.VMEM((3, CH, f2), jnp.float32),
            pltpu.VMEM_SHARED((NP, f2), jnp.float32),
            pltpu.VMEM_SHARED((NP, f2), jnp.float32),
            pltpu.SemaphoreType.DMA,
            pltpu.SemaphoreType.DMA,
            pltpu.SemaphoreType.DMA,
        ],
    )
    def _sc_scatter(ei_hbm, y_hbm, out_hbm, idx, rows, y_sh,
                    acc_sh, sem_i, sem_g, sem_s):
        cid = lax.axis_index("c")
        sid = lax.axis_index("s")
        sl = pl.ds(sid * RPT, RPT)
        base = sid * EPT
        ks = jnp.where(sid == NS - 1, KSL, KS)

        def load_idx(j, slot, sync=False):
            s_cp = pltpu.make_async_copy(
                ei_hbm.at[0, pl.ds(base + j * CH, CH)], idx.at[slot, 0],
                sem_i)
            d_cp = pltpu.make_async_copy(
                ei_hbm.at[1, pl.ds(base + j * CH, CH)], idx.at[slot, 1],
                sem_i)
            s_cp.start()
            d_cp.start()
            if sync:
                s_cp.wait()
                d_cp.wait()

        def wait_idx(j, slot):
            pltpu.make_async_copy(
                ei_hbm.at[0, pl.ds(base + j * CH, CH)], idx.at[slot, 0],
                sem_i).wait()
            pltpu.make_async_copy(
                ei_hbm.at[1, pl.ds(base + j * CH, CH)], idx.at[slot, 1],
                sem_i).wait()

        pltpu.sync_copy(y_hbm.at[cid, sl], y_sh.at[sl])
        pltpu.sync_copy(y_hbm.at[cid, sl], acc_sh.at[sl])
        load_idx(0, 0, sync=True)
        plsc.subcore_barrier()

        pltpu.async_copy(y_sh.at[idx.at[0, 0]], rows.at[0], sem_g)
        load_idx(1, 1)

        def body(j, carry):
            rs = lax.rem(j, 3)
            ds = lax.rem(j, 4)
            pltpu.make_async_copy(y_sh.at[idx.at[ds, 0]], rows.at[rs],
                                  sem_g).wait()
            pltpu.async_copy(rows.at[rs], acc_sh.at[idx.at[ds, 1]], sem_s,
                             add=True)

            @pl.when(j >= 2)
            def _drain():
                pltpu.make_async_copy(rows.at[lax.rem(j + 1, 3)],
                                      acc_sh.at[idx.at[lax.rem(j + 2, 4), 1]],
                                      sem_s).wait()

            @pl.when(j + 1 < ks)
            def _pf():
                wait_idx(j + 1, lax.rem(j + 1, 4))
                pltpu.async_copy(y_sh.at[idx.at[lax.rem(j + 1, 4), 0]],
                                 rows.at[lax.rem(j + 1, 3)], sem_g)

            @pl.when(j + 2 < ks)
            def _pfi():
                load_idx(j + 2, lax.rem(j + 2, 4))

            return carry

        lax.fori_loop(0, ks, body, 0)
        pltpu.make_async_copy(rows.at[lax.rem(ks - 2, 3)],
                              acc_sh.at[idx.at[lax.rem(ks - 2, 4), 1]],
                              sem_s).wait()
        pltpu.make_async_copy(rows.at[lax.rem(ks - 1, 3)],
                              acc_sh.at[idx.at[lax.rem(ks - 1, 4), 1]],
                              sem_s).wait()
        plsc.subcore_barrier()
        pltpu.sync_copy(acc_sh.at[sl], out_hbm.at[cid, sl])

    return _sc_scatter


_sc_scatter_h = _make_sc_scatter(F_HID // 2)
_sc_scatter_o = _make_sc_scatter(F_OUT // 2)


def _tc1_body(x_ref, w_ref, deg_ref, y_ref):
    dis = lax.rsqrt(deg_ref[0, :] + deg_ref[1, :] + 1.0)
    xw = jnp.dot(x_ref[...], w_ref[...], preferred_element_type=jnp.float32)
    y = xw * dis[:, None]
    y_ref[0] = y[:, :F_HID // 2]
    y_ref[1] = y[:, F_HID // 2:]


def _tc1(xp, W1, deg):
    return pl.pallas_call(
        _tc1_body,
        grid=(NP // BR,),
        in_specs=[
            pl.BlockSpec((BR, F_IN), lambda i: (i, 0)),
            pl.BlockSpec((F_IN, F_HID), lambda i: (0, 0)),
            pl.BlockSpec((NC, BR), lambda i: (0, i)),
        ],
        out_specs=pl.BlockSpec((NC, BR, F_HID // 2), lambda i: (0, i, 0)),
        out_shape=jax.ShapeDtypeStruct((NC, NP, F_HID // 2), jnp.float32),
    )(xp, W1, deg)


def _tc2_body(a_ref, deg_ref, b1_ref, w2_ref, y2_ref):
    i = pl.program_id(0)
    dis = lax.rsqrt(deg_ref[0, :] + deg_ref[1, :] + 1.0)
    agg = jnp.concatenate([a_ref[0], a_ref[1]], axis=1)
    h = jnp.maximum(agg * dis[:, None] + b1_ref[...][None, :], 0.0)
    row = i * BR + lax.broadcasted_iota(jnp.int32, (BR, 1), 0)
    h = jnp.where(row < N, h, 0.0)
    hw = jnp.dot(h, w2_ref[...], preferred_element_type=jnp.float32)
    y2 = hw * dis[:, None]
    y2_ref[0] = y2[:, :F_OUT // 2]
    y2_ref[1] = y2[:, F_OUT // 2:]


def _tc2(a, deg, b1, W2):
    return pl.pallas_call(
        _tc2_body,
        grid=(NP // BR,),
        in_specs=[
            pl.BlockSpec((NC, BR, F_HID // 2), lambda i: (0, i, 0)),
            pl.BlockSpec((NC, BR), lambda i: (0, i)),
            pl.BlockSpec((F_HID,), lambda i: (0,)),
            pl.BlockSpec((F_HID, F_OUT), lambda i: (0, 0)),
        ],
        out_specs=pl.BlockSpec((NC, BR, F_OUT // 2), lambda i: (0, i, 0)),
        out_shape=jax.ShapeDtypeStruct((NC, NP, F_OUT // 2), jnp.float32),
    )(a, deg, b1, W2)


def _tc3_body(c_ref, deg_ref, b2_ref, o_ref):
    dis = lax.rsqrt(deg_ref[0, :] + deg_ref[1, :] + 1.0)
    agg = jnp.concatenate([c_ref[0], c_ref[1]], axis=1)
    logits = agg * dis[:, None] + b2_ref[...][None, :]
    m = jnp.max(logits, axis=1, keepdims=True)
    e = jnp.exp(logits - m)
    o_ref[...] = e / jnp.sum(e, axis=1, keepdims=True)


def _tc3(c, deg, b2):
    return pl.pallas_call(
        _tc3_body,
        grid=(NP // BR,),
        in_specs=[
            pl.BlockSpec((NC, BR, F_OUT // 2), lambda i: (0, i, 0)),
            pl.BlockSpec((NC, BR), lambda i: (0, i)),
            pl.BlockSpec((F_OUT,), lambda i: (0,)),
        ],
        out_specs=pl.BlockSpec((BR, F_OUT), lambda i: (i, 0)),
        out_shape=jax.ShapeDtypeStruct((NP, F_OUT), jnp.float32),
    )(c, deg, b2)


def kernel(x, edge_index, W1, b1, W2, b2):
    ei = edge_index.astype(jnp.int32)
    xp = jnp.pad(x, ((0, NP - N), (0, 0)))
    zeros_np = jnp.zeros((NP,), jnp.float32)

    deg = _sc_degree(ei, zeros_np)
    y1 = _tc1(xp, W1, deg)
    a = _sc_scatter_h(ei, y1)
    y2 = _tc2(a, deg, b1, W2)
    c = _sc_scatter_o(ei, y2)
    out = _tc3(c, deg, b2)
    return out[:N]


# speedup vs baseline: 1.0725x; 1.0725x over previous
import functools

import jax
import jax.numpy as jnp
from jax import lax
from jax.experimental import pallas as pl
from jax.experimental.pallas import tpu as pltpu
from jax.experimental.pallas import tpu_sc as plsc

N = 10000
NP = 10240
F_IN = 128
F_HID = 128
F_OUT = 64
E = 320000
NC = 2
NS = 16
CH = 128
EPT = (E // (NS * CH)) * CH
KS = EPT // CH
KSL = (E - (NS - 1) * EPT) // CH
EPW = (E // (NC * NS * CH)) * CH
KW = EPW // CH
KWL = (E - (NC * NS - 1) * EPW) // CH
RPT = NP // NS
BR = NP // 8

_mesh = plsc.VectorSubcoreMesh(core_axis_name="c", subcore_axis_name="s")


@functools.partial(
    pl.kernel, mesh=_mesh,
    compiler_params=pltpu.CompilerParams(use_tc_tiling_on_sc=False),
    out_type=jax.ShapeDtypeStruct((NC, NP), jnp.float32),
    scratch_types=[
        pltpu.VMEM((KWL, CH), jnp.int32),
        pltpu.VMEM((CH,), jnp.float32),
        pltpu.VMEM_SHARED((NP,), jnp.float32),
    ],
)
def _sc_degree(ei_hbm, zeros_hbm, out_hbm, idx_v, ones_v, acc_sh):
    cid = lax.axis_index("c")
    sid = lax.axis_index("s")
    w = cid * NS + sid
    kw = jnp.where(w == NC * NS - 1, KWL, KW)
    for i in range(CH // 16):
        ones_v[pl.ds(i * 16, 16)] = jnp.ones((16,), jnp.float32)
    pltpu.sync_copy(zeros_hbm.at[pl.ds(sid * RPT, RPT)],
                    acc_sh.at[pl.ds(sid * RPT, RPT)])

    @pl.when(w == NC * NS - 1)
    def _tail():
        pltpu.sync_copy(ei_hbm.at[1, pl.ds(w * KW, KWL)], idx_v)

    @pl.when(w != NC * NS - 1)
    def _full():
        pltpu.sync_copy(ei_hbm.at[1, pl.ds(w * KW, KW)],
                        idx_v.at[pl.ds(0, KW)])

    plsc.subcore_barrier()

    def body(j, carry):
        pltpu.sync_copy(ones_v, acc_sh.at[idx_v.at[j]], add=True)
        return carry

    lax.fori_loop(0, kw, body, 0)
    plsc.subcore_barrier()
    pltpu.sync_copy(acc_sh.at[pl.ds(sid * RPT, RPT)],
                    out_hbm.at[cid, pl.ds(sid * RPT, RPT)])


def _make_sc_scatter(f2):
    @functools.partial(
        pl.kernel, mesh=_mesh,
        compiler_params=pltpu.CompilerParams(use_tc_tiling_on_sc=False),
        out_type=jax.ShapeDtypeStruct((NC, NP, f2), jnp.float32),
        scratch_types=[
            pltpu.VMEM((4, 2, CH), jnp.int32),
            pltpu.VMEM((3, CH, f2), jnp.float32),
            pltpu.VMEM_SHARED((NP, f2), jnp.float32),
            pltpu.VMEM_SHARED((NP, f2), jnp.float32),
            pltpu.SemaphoreType.DMA,
            pltpu.SemaphoreType.DMA,
            pltpu.SemaphoreType.DMA,
        ],
    )
    def _sc_scatter(ei_hbm, y_hbm, out_hbm, idx, rows, y_sh,
                    acc_sh, sem_i, sem_g, sem_s):
        cid = lax.axis_index("c")
        sid = lax.axis_index("s")
        sl = pl.ds(sid * RPT, RPT)
        cb = sid * KS
        ks = jnp.where(sid == NS - 1, KSL, KS)

        def load_idx(j, slot, sync=False):
            s_cp = pltpu.make_async_copy(ei_hbm.at[0, cb + j],
                                         idx.at[slot, 0], sem_i)
            d_cp = pltpu.make_async_copy(ei_hbm.at[1, cb + j],
                                         idx.at[slot, 1], sem_i)
            s_cp.start()
            d_cp.start()
            if sync:
                s_cp.wait()
                d_cp.wait()

        def wait_idx(j, slot):
            pltpu.make_async_copy(ei_hbm.at[0, cb + j], idx.at[slot, 0],
                                  sem_i).wait()
            pltpu.make_async_copy(ei_hbm.at[1, cb + j], idx.at[slot, 1],
                                  sem_i).wait()

        pltpu.sync_copy(y_hbm.at[cid, sl], y_sh.at[sl])
        pltpu.sync_copy(y_hbm.at[cid, sl], acc_sh.at[sl])
        load_idx(0, 0, sync=True)
        plsc.subcore_barrier()

        pltpu.async_copy(y_sh.at[idx.at[0, 0]], rows.at[0], sem_g)
        load_idx(1, 1)

        def body(j, carry):
            rs = lax.rem(j, 3)
            ds = lax.rem(j, 4)
            pltpu.make_async_copy(y_sh.at[idx.at[ds, 0]], rows.at[rs],
                                  sem_g).wait()
            pltpu.async_copy(rows.at[rs], acc_sh.at[idx.at[ds, 1]], sem_s,
                             add=True)

            @pl.when(j >= 2)
            def _drain():
                pltpu.make_async_copy(rows.at[lax.rem(j + 1, 3)],
                                      acc_sh.at[idx.at[lax.rem(j + 2, 4), 1]],
                                      sem_s).wait()

            @pl.when(j + 1 < ks)
            def _pf():
                wait_idx(j + 1, lax.rem(j + 1, 4))
                pltpu.async_copy(y_sh.at[idx.at[lax.rem(j + 1, 4), 0]],
                                 rows.at[lax.rem(j + 1, 3)], sem_g)

            @pl.when(j + 2 < ks)
            def _pfi():
                load_idx(j + 2, lax.rem(j + 2, 4))

            return carry

        lax.fori_loop(0, ks, body, 0)
        pltpu.make_async_copy(rows.at[lax.rem(ks - 2, 3)],
                              acc_sh.at[idx.at[lax.rem(ks - 2, 4), 1]],
                              sem_s).wait()
        pltpu.make_async_copy(rows.at[lax.rem(ks - 1, 3)],
                              acc_sh.at[idx.at[lax.rem(ks - 1, 4), 1]],
                              sem_s).wait()
        plsc.subcore_barrier()
        pltpu.sync_copy(acc_sh.at[sl], out_hbm.at[cid, sl])

    return _sc_scatter


_sc_scatter_h = _make_sc_scatter(F_HID // 2)
_sc_scatter_o = _make_sc_scatter(F_OUT // 2)


def _tc1_body(x_ref, w_ref, deg_ref, y_ref):
    dis = lax.rsqrt(deg_ref[0, :] + deg_ref[1, :] + 1.0)
    xw = jnp.dot(x_ref[...], w_ref[...], preferred_element_type=jnp.float32)
    y = xw * dis[:, None]
    y_ref[0] = y[:, :F_HID // 2]
    y_ref[1] = y[:, F_HID // 2:]


def _tc1(xp, W1, deg):
    return pl.pallas_call(
        _tc1_body,
        grid=(NP // BR,),
        in_specs=[
            pl.BlockSpec((BR, F_IN), lambda i: (i, 0)),
            pl.BlockSpec((F_IN, F_HID), lambda i: (0, 0)),
            pl.BlockSpec((NC, BR), lambda i: (0, i)),
        ],
        out_specs=pl.BlockSpec((NC, BR, F_HID // 2), lambda i: (0, i, 0)),
        out_shape=jax.ShapeDtypeStruct((NC, NP, F_HID // 2), jnp.float32),
    )(xp, W1, deg)


def _tc2_body(a_ref, deg_ref, b1_ref, w2_ref, y2_ref):
    i = pl.program_id(0)
    dis = lax.rsqrt(deg_ref[0, :] + deg_ref[1, :] + 1.0)
    agg = jnp.concatenate([a_ref[0], a_ref[1]], axis=1)
    h = jnp.maximum(agg * dis[:, None] + b1_ref[...][None, :], 0.0)
    row = i * BR + lax.broadcasted_iota(jnp.int32, (BR, 1), 0)
    h = jnp.where(row < N, h, 0.0)
    hw = jnp.dot(h, w2_ref[...], preferred_element_type=jnp.float32)
    y2 = hw * dis[:, None]
    y2_ref[0] = y2[:, :F_OUT // 2]
    y2_ref[1] = y2[:, F_OUT // 2:]


def _tc2(a, deg, b1, W2):
    return pl.pallas_call(
        _tc2_body,
        grid=(NP // BR,),
        in_specs=[
            pl.BlockSpec((NC, BR, F_HID // 2), lambda i: (0, i, 0)),
            pl.BlockSpec((NC, BR), lambda i: (0, i)),
            pl.BlockSpec((F_HID,), lambda i: (0,)),
            pl.BlockSpec((F_HID, F_OUT), lambda i: (0, 0)),
        ],
        out_specs=pl.BlockSpec((NC, BR, F_OUT // 2), lambda i: (0, i, 0)),
        out_shape=jax.ShapeDtypeStruct((NC, NP, F_OUT // 2), jnp.float32),
    )(a, deg, b1, W2)


def _tc3_body(c_ref, deg_ref, b2_ref, o_ref):
    dis = lax.rsqrt(deg_ref[0, :] + deg_ref[1, :] + 1.0)
    agg = jnp.concatenate([c_ref[0], c_ref[1]], axis=1)
    logits = agg * dis[:, None] + b2_ref[...][None, :]
    m = jnp.max(logits, axis=1, keepdims=True)
    e = jnp.exp(logits - m)
    o_ref[...] = e / jnp.sum(e, axis=1, keepdims=True)


def _tc3(c, deg, b2):
    return pl.pallas_call(
        _tc3_body,
        grid=(NP // BR,),
        in_specs=[
            pl.BlockSpec((NC, BR, F_OUT // 2), lambda i: (0, i, 0)),
            pl.BlockSpec((NC, BR), lambda i: (0, i)),
            pl.BlockSpec((F_OUT,), lambda i: (0,)),
        ],
        out_specs=pl.BlockSpec((BR, F_OUT), lambda i: (i, 0)),
        out_shape=jax.ShapeDtypeStruct((NP, F_OUT), jnp.float32),
    )(c, deg, b2)


def kernel(x, edge_index, W1, b1, W2, b2):
    ei3 = edge_index.astype(jnp.int32).reshape(2, E // CH, CH)
    xp = jnp.pad(x, ((0, NP - N), (0, 0)))
    zeros_np = jnp.zeros((NP,), jnp.float32)

    deg = _sc_degree(ei3, zeros_np)
    y1 = _tc1(xp, W1, deg)
    a = _sc_scatter_h(ei3, y1)
    y2 = _tc2(a, deg, b1, W2)
    c = _sc_scatter_o(ei3, y2)
    out = _tc3(c, deg, b2)
    return out[:N]
